# baseline (speedup 1.0000x reference)
import jax
import jax.numpy as jnp
from jax import lax
from jax.experimental import pallas as pl
from jax.experimental.pallas import tpu as pltpu
from jax.experimental.pallas import tpu_sc as plsc

NROI = 64
NPTS = 20000
NFEAT = 16
OX = OY = OZ = 12
NVOX = OX * OY * OZ
MV = 128
NWORK = 32
RPW = NROI // NWORK
PVEC = NPTS // 16


def _broadcast_params(prm):
  return [jnp.broadcast_to(prm[k], (16,)) for k in range(13)]


BLKV = 10
NBLK = PVEC // BLKV


def _body(xs, ys, zs, roip, feat, outf, outsel,
          xs_v, ys_v, zs_v, roip_v, pool0_v, pool1_v, occ0_v, occ1_v,
          sel_v, row_v, outf_v, outsel_v):
  cid = lax.axis_index("c")
  sid = lax.axis_index("s")
  wid = sid * 2 + cid

  pltpu.sync_copy(xs, xs_v)
  pltpu.sync_copy(ys, ys_v)
  pltpu.sync_copy(zs, zs_v)
  pltpu.sync_copy(roip, roip_v)

  n0 = wid * RPW
  n1 = n0 + 1
  (cx0, cy0, cz0, cc0, ss0, dx0, dy0, dz0, gx0, gy0, gz0, hx0, hy0
   ) = _broadcast_params(roip_v[pl.ds(n0 * 16, 16)])
  (cx1, cy1, cz1, cc1, ss1, dx1, dy1, dz1, gx1, gy1, gz1, hx1, hy1
   ) = _broadcast_params(roip_v[pl.ds(n1 * 16, 16)])

  zero16 = jnp.zeros((16,), jnp.int32)
  neg_inf = jnp.full((16,), -jnp.inf, jnp.float32)
  ones16 = jnp.ones((16,), jnp.int32)
  zero16f = jnp.zeros((16,), jnp.float32)

  def zocc(i, _):
    for u in range(2):
      occ0_v[pl.ds((i * 2 + u) * 16, 16)] = zero16
      occ1_v[pl.ds((i * 2 + u) * 16, 16)] = zero16
    return 0
  lax.fori_loop(0, NVOX // 32, zocc, 0)

  def zpool(i, _):
    for u in range(2):
      pool0_v[pl.ds((i * 2 + u) * 16, 16)] = neg_inf
      pool1_v[pl.ds((i * 2 + u) * 16, 16)] = neg_inf
    return 0
  lax.fori_loop(0, NVOX * NFEAT // 32, zpool, 0)

  def masks_for(i):
    x = xs_v[pl.ds(i * 16, 16)]
    y = ys_v[pl.ds(i * 16, 16)]
    z = zs_v[pl.ds(i * 16, 16)]
    sx0 = x - cx0
    sy0 = y - cy0
    xl0 = sx0 * cc0 - sy0 * ss0 + hx0
    yl0 = sx0 * ss0 + sy0 * cc0 + hy0
    zl0 = z - cz0
    inb0 = (((xl0 >= zero16f) & (xl0 < dx0))
            & ((yl0 >= zero16f) & (yl0 < dy0))
            & ((zl0 >= zero16f) & (zl0 < dz0)))
    sx1 = x - cx1
    sy1 = y - cy1
    xl1 = sx1 * cc1 - sy1 * ss1 + hx1
    yl1 = sx1 * ss1 + sy1 * cc1 + hy1
    zl1 = z - cz1
    inb1 = (((xl1 >= zero16f) & (xl1 < dx1))
            & ((yl1 >= zero16f) & (yl1 < dy1))
            & ((zl1 >= zero16f) & (zl1 < dz1)))
    return inb0, xl0, yl0, zl0, inb1, xl1, yl1, zl1

  def vec_body(i):
    inb0, xl0, yl0, zl0, inb1, xl1, yl1, zl1 = masks_for(i)
    hit = plsc.all_reduce_population_count(inb0 | inb1)[0]

    @pl.when(hit > 0)
    def _():
      for inb, xl, yl, zl, gx, gy, gz, occ_v, pool_v in (
          (inb0, xl0, yl0, zl0, gx0, gy0, gz0, occ0_v, pool0_v),
          (inb1, xl1, yl1, zl1, gx1, gy1, gz1, occ1_v, pool1_v)):
        cnt = plsc.all_reduce_population_count(inb)[0]

        @pl.when(cnt > 0)
        def _():
          mi = inb.astype(jnp.int32)
          vx = jnp.clip((xl / gx).astype(jnp.int32), 0, OX - 1)
          vy = jnp.clip((yl / gy).astype(jnp.int32), 0, OY - 1)
          vz = jnp.clip((zl / gz).astype(jnp.int32), 0, OZ - 1)
          vox = (vx * OY + vy) * OZ + vz
          plsc.store_scatter(occ_v, [vox], ones16, mask=inb)
          for l in range(16):
            @pl.when(mi[l] != 0)
            def _():
              base = vox[l] * NFEAT
              pltpu.sync_copy(feat.at[pl.ds((i * 16 + l) * NFEAT, 16)],
                              row_v)
              fr = row_v[...]
              pool_v[pl.ds(base, 16)] = jnp.maximum(
                  pool_v[pl.ds(base, 16)], fr)

  def blockfn(j, _):
    i0 = j * BLKV
    m0, _, _, _, m1, _, _, _ = masks_for(i0)
    acc = m0 | m1
    for vl in range(1, BLKV):
      m0, _, _, _, m1, _, _, _ = masks_for(i0 + vl)
      acc = acc | (m0 | m1)
    anyhit = plsc.all_reduce_population_count(acc)[0]

    @pl.when(anyhit > 0)
    def _():
      for vl in range(BLKV):
        vec_body(i0 + vl)
    return 0
  lax.fori_loop(0, NBLK, blockfn, 0)

  jv16 = lax.iota(jnp.int32, 16)

  for n, pool_v, occ_v in ((n0, pool0_v, occ0_v), (n1, pool1_v, occ1_v)):
    def compact(i, pos):
      ov = occ_v[pl.ds(i * 16, 16)]
      m = ov != 0
      mi2 = m.astype(jnp.int32)
      ids = lax.iota(jnp.int32, 16) + i * 16
      tgt = pos + (plsc.cumsum(mi2) - mi2)
      plsc.store_scatter(sel_v, [tgt], ids, mask=m)
      return pos + plsc.all_reduce_population_count(m)[0]
    n_ne = lax.fori_loop(0, NVOX // 16, compact, 0)

    def emit(jv, _):
      selvec = sel_v[pl.ds(jv * 16, 16)]
      validv = (jv16 + jv * 16) < n_ne
      validi = validv.astype(jnp.int32)
      safe = jnp.where(validv, selvec, 0)
      outsel_v[pl.ds(jv * 16, 16)] = jnp.where(validv, selvec, -1)
      for l in range(16):
        rowd = pool_v[pl.ds(safe[l] * NFEAT, 16)]
        outf_v[pl.ds((jv * 16 + l) * 16, 16)] = jnp.where(
            validi[l] != 0, rowd, 0.0)
      return 0
    lax.fori_loop(0, MV // 16, emit, 0)

    pltpu.sync_copy(outf_v, outf.at[n])
    pltpu.sync_copy(outsel_v, outsel.at[n])


@jax.jit
def _run(xs, ys, zs, roip, feat):
  f = pl.kernel(
      _body,
      out_type=(jax.ShapeDtypeStruct((NROI, MV * NFEAT), jnp.float32),
                jax.ShapeDtypeStruct((NROI, MV), jnp.int32)),
      mesh=plsc.VectorSubcoreMesh(core_axis_name="c", subcore_axis_name="s"),
      compiler_params=pltpu.CompilerParams(needs_layout_passes=False),
      scratch_types=[
          pltpu.VMEM((NPTS,), jnp.float32),
          pltpu.VMEM((NPTS,), jnp.float32),
          pltpu.VMEM((NPTS,), jnp.float32),
          pltpu.VMEM((NROI * 16,), jnp.float32),
          pltpu.VMEM((NVOX * NFEAT,), jnp.float32),
          pltpu.VMEM((NVOX * NFEAT,), jnp.float32),
          pltpu.VMEM((NVOX,), jnp.int32),
          pltpu.VMEM((NVOX,), jnp.int32),
          pltpu.VMEM((NVOX + 16,), jnp.int32),
          pltpu.VMEM((16,), jnp.float32),
          pltpu.VMEM((MV * NFEAT,), jnp.float32),
          pltpu.VMEM((MV,), jnp.int32),
      ],
  )
  return f(xs, ys, zs, roip, feat)


def kernel(rois, pts, pts_feature):
  centers = rois[:, 0:3]
  dims = rois[:, 3:6]
  rz = rois[:, 6]
  cc = jnp.cos(-rz)
  ss = jnp.sin(-rz)
  pad = jnp.zeros((NROI,), jnp.float32)
  roip = jnp.stack([
      centers[:, 0], centers[:, 1], centers[:, 2],
      cc, ss,
      dims[:, 0], dims[:, 1], dims[:, 2],
      dims[:, 0] / OX, dims[:, 1] / OY, dims[:, 2] / OZ,
      dims[:, 0] * 0.5, dims[:, 1] * 0.5,
      pad, pad, pad,
  ], axis=1).astype(jnp.float32).reshape(NROI * 16)
  xs = pts[:, 0].astype(jnp.float32)
  ys = pts[:, 1].astype(jnp.float32)
  zs = pts[:, 2].astype(jnp.float32)
  featout, selout = _run(
      xs, ys, zs, roip,
      pts_feature.astype(jnp.float32).reshape(NPTS * NFEAT))
  pooled_features = featout.reshape(NROI, MV, NFEAT)
  valid = selout >= 0
  svx = selout // (OY * OZ)
  rem = selout % (OY * OZ)
  svy = rem // OZ
  svz = rem % OZ
  coors = jnp.stack([svx, svy, svz], axis=-1).astype(jnp.int32)
  pooled_coors = jnp.where(valid[..., None], coors, -1)
  return pooled_features, pooled_coors


# speedup vs baseline: 1.6746x; 1.6746x over previous
import jax
import jax.numpy as jnp
from jax import lax
from jax.experimental import pallas as pl
from jax.experimental.pallas import tpu as pltpu
from jax.experimental.pallas import tpu_sc as plsc

NROI = 64
NPTS = 20000
NFEAT = 16
OX = OY = OZ = 12
NVOX = OX * OY * OZ
MV = 128
NWORK = 32
RPW = NROI // NWORK
PVEC = NPTS // 16


def _broadcast_params(prm):
  return [jnp.broadcast_to(prm[k], (16,)) for k in range(13)]


BLKV = 10
NBLK = PVEC // BLKV


def _body(xs, ys, zs, roip, feat, outf, outsel,
          xs_v, ys_v, zs_v, roip_v, pool0_v, pool1_v, occ0_v, occ1_v,
          sel_v, row_v, outf_v, outsel_v):
  cid = lax.axis_index("c")
  sid = lax.axis_index("s")
  wid = sid * 2 + cid

  pltpu.sync_copy(xs, xs_v)
  pltpu.sync_copy(ys, ys_v)
  pltpu.sync_copy(zs, zs_v)
  pltpu.sync_copy(roip, roip_v)

  n0 = wid * RPW
  n1 = n0 + 1
  (cx0, cy0, cz0, cc0, ss0, dx0, dy0, dz0, gx0, gy0, gz0, hx0, hy0
   ) = _broadcast_params(roip_v[pl.ds(n0 * 16, 16)])
  (cx1, cy1, cz1, cc1, ss1, dx1, dy1, dz1, gx1, gy1, gz1, hx1, hy1
   ) = _broadcast_params(roip_v[pl.ds(n1 * 16, 16)])

  zero16 = jnp.zeros((16,), jnp.int32)
  neg_inf = jnp.full((16,), -jnp.inf, jnp.float32)
  ones16 = jnp.ones((16,), jnp.int32)
  zero16f = jnp.zeros((16,), jnp.float32)

  def zocc(i, _):
    for u in range(2):
      occ0_v[pl.ds((i * 2 + u) * 16, 16)] = zero16
      occ1_v[pl.ds((i * 2 + u) * 16, 16)] = zero16
    return 0
  lax.fori_loop(0, NVOX // 32, zocc, 0)

  def zpool(i, _):
    for u in range(2):
      pool0_v[pl.ds((i * 2 + u) * 16, 16)] = neg_inf
      pool1_v[pl.ds((i * 2 + u) * 16, 16)] = neg_inf
    return 0
  lax.fori_loop(0, NVOX * NFEAT // 32, zpool, 0)

  def masks_for(i):
    x = xs_v[pl.ds(i * 16, 16)]
    y = ys_v[pl.ds(i * 16, 16)]
    z = zs_v[pl.ds(i * 16, 16)]
    sx0 = x - cx0
    sy0 = y - cy0
    xl0 = sx0 * cc0 - sy0 * ss0 + hx0
    yl0 = sx0 * ss0 + sy0 * cc0 + hy0
    zl0 = z - cz0
    inb0 = (((xl0 >= zero16f) & (xl0 < dx0))
            & ((yl0 >= zero16f) & (yl0 < dy0))
            & ((zl0 >= zero16f) & (zl0 < dz0)))
    sx1 = x - cx1
    sy1 = y - cy1
    xl1 = sx1 * cc1 - sy1 * ss1 + hx1
    yl1 = sx1 * ss1 + sy1 * cc1 + hy1
    zl1 = z - cz1
    inb1 = (((xl1 >= zero16f) & (xl1 < dx1))
            & ((yl1 >= zero16f) & (yl1 < dy1))
            & ((zl1 >= zero16f) & (zl1 < dz1)))
    return inb0, xl0, yl0, zl0, inb1, xl1, yl1, zl1

  def vec_body(i):
    inb0, xl0, yl0, zl0, inb1, xl1, yl1, zl1 = masks_for(i)
    hit = plsc.all_reduce_population_count(inb0 | inb1)[0]

    @pl.when(hit > 0)
    def _():
      for inb, xl, yl, zl, gx, gy, gz, occ_v, pool_v in (
          (inb0, xl0, yl0, zl0, gx0, gy0, gz0, occ0_v, pool0_v),
          (inb1, xl1, yl1, zl1, gx1, gy1, gz1, occ1_v, pool1_v)):
        cnt = plsc.all_reduce_population_count(inb)[0]

        @pl.when(cnt > 0)
        def _():
          mi = inb.astype(jnp.int32)
          vx = jnp.clip((xl / gx).astype(jnp.int32), 0, OX - 1)
          vy = jnp.clip((yl / gy).astype(jnp.int32), 0, OY - 1)
          vz = jnp.clip((zl / gz).astype(jnp.int32), 0, OZ - 1)
          vox = (vx * OY + vy) * OZ + vz
          plsc.store_scatter(occ_v, [vox], ones16, mask=inb)
          for l in range(16):
            @pl.when(mi[l] != 0)
            def _():
              base = vox[l] * NFEAT
              pltpu.sync_copy(feat.at[pl.ds((i * 16 + l) * NFEAT, 16)],
                              row_v)
              fr = row_v[...]
              pool_v[pl.ds(base, 16)] = jnp.maximum(
                  pool_v[pl.ds(base, 16)], fr)

  def sweep(i, _):
    vec_body(i)
    return 0
  lax.fori_loop(0, PVEC, sweep, 0)

  jv16 = lax.iota(jnp.int32, 16)

  for n, pool_v, occ_v in ((n0, pool0_v, occ0_v), (n1, pool1_v, occ1_v)):
    def compact(i, pos):
      ov = occ_v[pl.ds(i * 16, 16)]
      m = ov != 0
      mi2 = m.astype(jnp.int32)
      ids = lax.iota(jnp.int32, 16) + i * 16
      tgt = pos + (plsc.cumsum(mi2) - mi2)
      plsc.store_scatter(sel_v, [tgt], ids, mask=m)
      return pos + plsc.all_reduce_population_count(m)[0]
    n_ne = lax.fori_loop(0, NVOX // 16, compact, 0)

    def emit(jv, _):
      selvec = sel_v[pl.ds(jv * 16, 16)]
      validv = (jv16 + jv * 16) < n_ne
      validi = validv.astype(jnp.int32)
      safe = jnp.where(validv, selvec, 0)
      outsel_v[pl.ds(jv * 16, 16)] = jnp.where(validv, selvec, -1)
      for l in range(16):
        rowd = pool_v[pl.ds(safe[l] * NFEAT, 16)]
        outf_v[pl.ds((jv * 16 + l) * 16, 16)] = jnp.where(
            validi[l] != 0, rowd, 0.0)
      return 0
    lax.fori_loop(0, MV // 16, emit, 0)

    pltpu.sync_copy(outf_v, outf.at[n])
    pltpu.sync_copy(outsel_v, outsel.at[n])


@jax.jit
def _run(xs, ys, zs, roip, feat):
  f = pl.kernel(
      _body,
      out_type=(jax.ShapeDtypeStruct((NROI, MV * NFEAT), jnp.float32),
                jax.ShapeDtypeStruct((NROI, MV), jnp.int32)),
      mesh=plsc.VectorSubcoreMesh(core_axis_name="c", subcore_axis_name="s"),
      compiler_params=pltpu.CompilerParams(needs_layout_passes=False),
      scratch_types=[
          pltpu.VMEM((NPTS,), jnp.float32),
          pltpu.VMEM((NPTS,), jnp.float32),
          pltpu.VMEM((NPTS,), jnp.float32),
          pltpu.VMEM((NROI * 16,), jnp.float32),
          pltpu.VMEM((NVOX * NFEAT,), jnp.float32),
          pltpu.VMEM((NVOX * NFEAT,), jnp.float32),
          pltpu.VMEM((NVOX,), jnp.int32),
          pltpu.VMEM((NVOX,), jnp.int32),
          pltpu.VMEM((NVOX + 16,), jnp.int32),
          pltpu.VMEM((16,), jnp.float32),
          pltpu.VMEM((MV * NFEAT,), jnp.float32),
          pltpu.VMEM((MV,), jnp.int32),
      ],
  )
  return f(xs, ys, zs, roip, feat)


def kernel(rois, pts, pts_feature):
  centers = rois[:, 0:3]
  dims = rois[:, 3:6]
  rz = rois[:, 6]
  cc = jnp.cos(-rz)
  ss = jnp.sin(-rz)
  pad = jnp.zeros((NROI,), jnp.float32)
  roip = jnp.stack([
      centers[:, 0], centers[:, 1], centers[:, 2],
      cc, ss,
      dims[:, 0], dims[:, 1], dims[:, 2],
      dims[:, 0] / OX, dims[:, 1] / OY, dims[:, 2] / OZ,
      dims[:, 0] * 0.5, dims[:, 1] * 0.5,
      pad, pad, pad,
  ], axis=1).astype(jnp.float32).reshape(NROI * 16)
  xs = pts[:, 0].astype(jnp.float32)
  ys = pts[:, 1].astype(jnp.float32)
  zs = pts[:, 2].astype(jnp.float32)
  featout, selout = _run(
      xs, ys, zs, roip,
      pts_feature.astype(jnp.float32).reshape(NPTS * NFEAT))
  pooled_features = featout.reshape(NROI, MV, NFEAT)
  valid = selout >= 0
  svx = selout // (OY * OZ)
  rem = selout % (OY * OZ)
  svy = rem // OZ
  svz = rem % OZ
  coors = jnp.stack([svx, svy, svz], axis=-1).astype(jnp.int32)
  pooled_coors = jnp.where(valid[..., None], coors, -1)
  return pooled_features, pooled_coors
